# Initial kernel scaffold; baseline (speedup 1.0000x reference)
import jax
import jax.numpy as jnp
from jax.experimental import pallas as pl


def kernel(inp_x, rotation, translation, inp):
    raise NotImplementedError("write your pallas kernel here")


# speedup vs baseline: 2619.2778x; 2619.2778x over previous
import functools

import jax
import jax.numpy as jnp
from jax import lax
from jax.experimental import pallas as pl
from jax.experimental.pallas import tpu as pltpu

NUM_FRAMES = 20
TOPK = 20
DIST_LAMBDA = 100.0
N_POINTS = 20000


def _rmat_body(rot6_ref, out_ref):
    a1x = rot6_ref[0:1, :]
    a1y = rot6_ref[1:2, :]
    a1z = rot6_ref[2:3, :]
    a2x = rot6_ref[3:4, :]
    a2y = rot6_ref[4:5, :]
    a2z = rot6_ref[5:6, :]
    inv1 = lax.rsqrt(a1x * a1x + a1y * a1y + a1z * a1z)
    b1x, b1y, b1z = a1x * inv1, a1y * inv1, a1z * inv1
    d = b1x * a2x + b1y * a2y + b1z * a2z
    ux, uy, uz = a2x - d * b1x, a2y - d * b1y, a2z - d * b1z
    inv2 = lax.rsqrt(ux * ux + uy * uy + uz * uz)
    b2x, b2y, b2z = ux * inv2, uy * inv2, uz * inv2
    b3x = b1y * b2z - b1z * b2y
    b3y = b1z * b2x - b1x * b2z
    b3z = b1x * b2y - b1y * b2x
    out_ref[0:1, :] = b1x
    out_ref[1:2, :] = b1y
    out_ref[2:3, :] = b1z
    out_ref[3:4, :] = b2x
    out_ref[4:5, :] = b2y
    out_ref[5:6, :] = b2z
    out_ref[6:7, :] = b3x
    out_ref[7:8, :] = b3y
    out_ref[8:9, :] = b3z


def _rmat_pallas(rot6_t, *, interpret=False):
    n = rot6_t.shape[1]
    return pl.pallas_call(
        _rmat_body,
        out_shape=jax.ShapeDtypeStruct((9, n), jnp.float32),
        interpret=interpret,
    )(rot6_t)


def kernel(inp_x, rotation, translation, inp):
    n = inp_x.shape[0]
    time_ind = jnp.round(inp[0, 3] * NUM_FRAMES).astype(jnp.int32)
    rot6 = lax.dynamic_index_in_dim(rotation, time_ind, axis=0, keepdims=False)
    trans = lax.dynamic_index_in_dim(translation, time_ind, axis=0, keepdims=False)
    r9 = _rmat_pallas(rot6.T)
    r_mat = r9.reshape(3, 3, n).transpose(2, 0, 1)
    return (r_mat, trans)
